# baseline (speedup 1.0000x reference)
import functools

import jax
import jax.numpy as jnp
import numpy as np
from jax import lax
from jax.experimental import pallas as pl
from jax.experimental.pallas import tpu as pltpu
from jax.experimental.pallas import tpu_sc as plsc

_REDUCTION_RATIO = 0.5
_NBUF = 12
_CHUNK = 8


@functools.cache
def _flat_row_indices(N: int, L: int) -> np.ndarray:
    _L = L - 1
    keep = int(_L * (1 - _REDUCTION_RATIO))
    with jax.ensure_compile_time_eval():
        scores = jax.random.uniform(jax.random.key(42), (N, _L), dtype=jnp.float32)
        pm = jnp.argsort(scores, axis=1)[:, :keep] + 1
        pm = jnp.sort(pm, axis=1)
        pm = np.asarray(pm)
    mask = np.concatenate([np.zeros((N, 1), pm.dtype), pm], axis=1)
    flat = np.arange(N, dtype=np.int64)[:, None] * L + mask
    return flat.astype(np.int32).reshape(-1)


def _make_sc_gather(R: int, D: int, n_workers: int, chunk: int):
    rows_per_w = R // n_workers
    nch = rows_per_w // chunk
    mesh = plsc.VectorSubcoreMesh(core_axis_name="c", subcore_axis_name="s")
    nc = 2

    @functools.partial(
        pl.kernel,
        out_type=jax.ShapeDtypeStruct((R, D), jnp.float32),
        mesh=mesh,
        scratch_types=[
            pltpu.VMEM((rows_per_w,), jnp.int32),
            pltpu.VMEM((_NBUF, chunk, D), jnp.float32),
        ]
        + [pltpu.SemaphoreType.DMA] * (2 * _NBUF),
    )
    def gather_kernel(x_hbm, idx_hbm, out_hbm, idx_v, bufs, *sems):
        wid = lax.axis_index("s") * nc + lax.axis_index("c")
        base = wid * rows_per_w
        pltpu.sync_copy(idx_hbm.at[pl.ds(base, rows_per_w)], idx_v)
        nbuf = _NBUF
        gsems = sems[:nbuf]
        wsems = sems[nbuf:]
        gathers = [None] * nbuf
        writes = [None] * nbuf
        for c in range(min(nbuf, nch)):
            gathers[c] = pltpu.async_copy(
                x_hbm.at[idx_v.at[pl.ds(c * chunk, chunk)]], bufs.at[c], gsems[c]
            )
        for c in range(nch):
            b = c % nbuf
            gathers[b].wait()
            writes[b] = pltpu.async_copy(
                bufs.at[b], out_hbm.at[pl.ds(base + c * chunk, chunk)], wsems[b]
            )
            nxt = c + nbuf
            if nxt < nch:
                writes[b].wait()
                writes[b] = None
                gathers[b] = pltpu.async_copy(
                    x_hbm.at[idx_v.at[pl.ds(nxt * chunk, chunk)]], bufs.at[b], gsems[b]
                )
        for b in range(nbuf):
            if writes[b] is not None:
                writes[b].wait()

    return gather_kernel


def kernel(x):
    N, L, D = x.shape
    keep1 = int((L - 1) * (1 - _REDUCTION_RATIO)) + 1
    R = N * keep1
    n_workers = 32
    chunk = _CHUNK
    assert R % (n_workers * chunk) == 0
    idx = _flat_row_indices(N, L)
    out2d = _make_sc_gather(R, D, n_workers, chunk)(
        x.reshape(N * L, D), jnp.asarray(idx)
    )
    return out2d.reshape(N, keep1, D)


# speedup vs baseline: 1.0004x; 1.0004x over previous
import functools

import jax
import jax.numpy as jnp
import numpy as np
from jax import lax
from jax.experimental import pallas as pl
from jax.experimental.pallas import tpu as pltpu
from jax.experimental.pallas import tpu_sc as plsc

_REDUCTION_RATIO = 0.5
_NBUF = 7
_CHUNK = 16


@functools.cache
def _flat_row_indices(N: int, L: int) -> np.ndarray:
    _L = L - 1
    keep = int(_L * (1 - _REDUCTION_RATIO))
    with jax.ensure_compile_time_eval():
        scores = jax.random.uniform(jax.random.key(42), (N, _L), dtype=jnp.float32)
        pm = jnp.argsort(scores, axis=1)[:, :keep] + 1
        pm = jnp.sort(pm, axis=1)
        pm = np.asarray(pm)
    mask = np.concatenate([np.zeros((N, 1), pm.dtype), pm], axis=1)
    flat = np.arange(N, dtype=np.int64)[:, None] * L + mask
    return flat.astype(np.int32).reshape(-1)


def _make_sc_gather(R: int, D: int, n_workers: int, chunk: int):
    rows_per_w = R // n_workers
    nch = rows_per_w // chunk
    mesh = plsc.VectorSubcoreMesh(core_axis_name="c", subcore_axis_name="s")
    nc = 2

    @functools.partial(
        pl.kernel,
        out_type=jax.ShapeDtypeStruct((R, D), jnp.float32),
        mesh=mesh,
        scratch_types=[
            pltpu.VMEM((rows_per_w,), jnp.int32),
            pltpu.VMEM((_NBUF, chunk, D), jnp.float32),
        ]
        + [pltpu.SemaphoreType.DMA] * (2 * _NBUF),
    )
    def gather_kernel(x_hbm, idx_hbm, out_hbm, idx_v, bufs, *sems):
        wid = lax.axis_index("s") * nc + lax.axis_index("c")
        base = wid * rows_per_w
        pltpu.sync_copy(idx_hbm.at[pl.ds(base, rows_per_w)], idx_v)
        nbuf = _NBUF
        gsems = sems[:nbuf]
        wsems = sems[nbuf:]
        gathers = [None] * nbuf
        writes = [None] * nbuf
        for c in range(min(nbuf, nch)):
            gathers[c] = pltpu.async_copy(
                x_hbm.at[idx_v.at[pl.ds(c * chunk, chunk)]], bufs.at[c], gsems[c]
            )
        for c in range(nch):
            b = c % nbuf
            gathers[b].wait()
            writes[b] = pltpu.async_copy(
                bufs.at[b], out_hbm.at[pl.ds(base + c * chunk, chunk)], wsems[b]
            )
            nxt = c + nbuf
            if nxt < nch:
                writes[b].wait()
                writes[b] = None
                gathers[b] = pltpu.async_copy(
                    x_hbm.at[idx_v.at[pl.ds(nxt * chunk, chunk)]], bufs.at[b], gsems[b]
                )
        for b in range(nbuf):
            if writes[b] is not None:
                writes[b].wait()

    return gather_kernel


def kernel(x):
    N, L, D = x.shape
    keep1 = int((L - 1) * (1 - _REDUCTION_RATIO)) + 1
    R = N * keep1
    n_workers = 32
    chunk = _CHUNK
    assert R % (n_workers * chunk) == 0
    idx = _flat_row_indices(N, L)
    out2d = _make_sc_gather(R, D, n_workers, chunk)(
        x.reshape(N * L, D), jnp.asarray(idx)
    )
    return out2d.reshape(N, keep1, D)


# speedup vs baseline: 1.0053x; 1.0049x over previous
import functools

import jax
import jax.numpy as jnp
import numpy as np
from jax import lax
from jax.experimental import pallas as pl
from jax.experimental.pallas import tpu as pltpu
from jax.experimental.pallas import tpu_sc as plsc

_REDUCTION_RATIO = 0.5
_NBUF = 6
_CHUNK = 16


@functools.cache
def _flat_row_indices(N: int, L: int) -> np.ndarray:
    _L = L - 1
    keep = int(_L * (1 - _REDUCTION_RATIO))
    with jax.ensure_compile_time_eval():
        scores = jax.random.uniform(jax.random.key(42), (N, _L), dtype=jnp.float32)
        pm = jnp.argsort(scores, axis=1)[:, :keep] + 1
        pm = jnp.sort(pm, axis=1)
        pm = np.asarray(pm)
    mask = np.concatenate([np.zeros((N, 1), pm.dtype), pm], axis=1)
    flat = np.arange(N, dtype=np.int64)[:, None] * L + mask
    return flat.astype(np.int32).reshape(-1)


def _make_sc_gather(R: int, D: int, n_workers: int, chunk: int):
    rows_per_w = R // n_workers
    nch = rows_per_w // chunk
    mesh = plsc.VectorSubcoreMesh(core_axis_name="c", subcore_axis_name="s")
    nc = 2

    @functools.partial(
        pl.kernel,
        out_type=jax.ShapeDtypeStruct((R, D), jnp.float32),
        mesh=mesh,
        scratch_types=[
            pltpu.VMEM((rows_per_w,), jnp.int32),
            pltpu.VMEM((_NBUF, chunk, D), jnp.float32),
        ]
        + [pltpu.SemaphoreType.DMA] * (2 * _NBUF),
    )
    def gather_kernel(x_hbm, idx_hbm, out_hbm, idx_v, bufs, *sems):
        wid = lax.axis_index("s") * nc + lax.axis_index("c")
        base = wid * rows_per_w
        pltpu.sync_copy(idx_hbm.at[pl.ds(base, rows_per_w)], idx_v)
        nbuf = _NBUF
        gsems = sems[:nbuf]
        wsems = sems[nbuf:]
        gathers = [None] * nbuf
        writes = [None] * nbuf
        for c in range(min(nbuf, nch)):
            gathers[c] = pltpu.async_copy(
                x_hbm.at[idx_v.at[pl.ds(c * chunk, chunk)]], bufs.at[c], gsems[c]
            )
        for c in range(nch):
            b = c % nbuf
            gathers[b].wait()
            writes[b] = pltpu.async_copy(
                bufs.at[b], out_hbm.at[pl.ds(base + c * chunk, chunk)], wsems[b]
            )
            nxt = c + nbuf
            if nxt < nch:
                writes[b].wait()
                writes[b] = None
                gathers[b] = pltpu.async_copy(
                    x_hbm.at[idx_v.at[pl.ds(nxt * chunk, chunk)]], bufs.at[b], gsems[b]
                )
        for b in range(nbuf):
            if writes[b] is not None:
                writes[b].wait()

    return gather_kernel


def kernel(x):
    N, L, D = x.shape
    keep1 = int((L - 1) * (1 - _REDUCTION_RATIO)) + 1
    R = N * keep1
    n_workers = 32
    chunk = _CHUNK
    assert R % (n_workers * chunk) == 0
    idx = _flat_row_indices(N, L)
    out2d = _make_sc_gather(R, D, n_workers, chunk)(
        x.reshape(N * L, D), jnp.asarray(idx)
    )
    return out2d.reshape(N, keep1, D)
